# Initial kernel scaffold; baseline (speedup 1.0000x reference)
import jax
import jax.numpy as jnp
from jax.experimental import pallas as pl


def kernel(obs, memory, W, b):
    raise NotImplementedError("write your pallas kernel here")


# speedup vs baseline: 1.4843x; 1.4843x over previous
import functools

import jax
import jax.numpy as jnp
from jax.experimental import pallas as pl
from jax.experimental.pallas import tpu as pltpu

TM = 2048


def _body(obsT_ref, W_ref, bT_ref, mem_ref, out_ref, embT_ref, q2_ref, run_ref,
          *, m_total, n_tiles, n_q, k_top):
    i = pl.program_id(0)

    @pl.when(i == 0)
    def _init():
        embT = jax.lax.dot_general(
            W_ref[...], obsT_ref[...], (((0,), (0,)), ((), ())),
            preferred_element_type=jnp.float32)
        embT = embT + bT_ref[...]
        embT_ref[...] = embT
        q2 = jnp.sum(embT * embT, axis=0, keepdims=True)
        q2_ref[...] = jnp.broadcast_to(q2, q2_ref.shape)
        run_ref[...] = jnp.full(run_ref.shape, jnp.inf, jnp.float32)

    mem = mem_ref[...]
    qm = jax.lax.dot_general(
        mem, embT_ref[...], (((1,), (0,)), ((), ())),
        preferred_element_type=jnp.float32)
    m2 = jnp.sum(mem * mem, axis=1, keepdims=True)
    d2 = m2 + q2_ref[0:1, :] - 2.0 * qm

    rows = jax.lax.broadcasted_iota(jnp.int32, d2.shape, 0)
    d2 = jnp.where(rows + i * TM < m_total, d2, jnp.inf)

    x = jnp.concatenate([d2, run_ref[...]], axis=0)
    xrows = jax.lax.broadcasted_iota(jnp.int32, x.shape, 0)
    mins = []
    for _ in range(k_top):
        mn = jnp.min(x, axis=0, keepdims=True)
        first = jnp.min(jnp.where(x == mn, xrows, x.shape[0]),
                        axis=0, keepdims=True)
        x = jnp.where(xrows == first, jnp.inf, x)
        mins.append(mn)
    pad = jnp.full((run_ref.shape[0] - k_top, n_q), jnp.inf, jnp.float32)
    newrun = jnp.concatenate(mins + [pad], axis=0)
    run_ref[...] = newrun

    @pl.when(i == n_tiles - 1)
    def _fin():
        top = newrun[0:k_top, :]
        d = jnp.sqrt(jnp.maximum(top, 0.0) + 1e-12)
        out_ref[0, 0] = jnp.sum(d) / (k_top * n_q)


@functools.partial(jax.jit, static_argnames=("interpret",))
def _novelty(obsT, memory, W, bT, interpret=False):
    m_total, d_dim = memory.shape
    n_q = obsT.shape[1]
    k_top = 5
    n_tiles = pl.cdiv(m_total, TM)
    out = pl.pallas_call(
        functools.partial(_body, m_total=m_total, n_tiles=n_tiles,
                          n_q=n_q, k_top=k_top),
        grid=(n_tiles,),
        in_specs=[
            pl.BlockSpec((obsT.shape[0], n_q), lambda i: (0, 0)),
            pl.BlockSpec(W.shape, lambda i: (0, 0)),
            pl.BlockSpec((d_dim, 1), lambda i: (0, 0)),
            pl.BlockSpec((TM, d_dim), lambda i: (i, 0)),
        ],
        out_specs=pl.BlockSpec(memory_space=pltpu.SMEM),
        out_shape=jax.ShapeDtypeStruct((1, 1), jnp.float32),
        scratch_shapes=[
            pltpu.VMEM((d_dim, n_q), jnp.float32),
            pltpu.VMEM((8, n_q), jnp.float32),
            pltpu.VMEM((8, n_q), jnp.float32),
        ],
        compiler_params=pltpu.CompilerParams(
            dimension_semantics=("arbitrary",)),
        interpret=interpret,
    )(obsT, W, bT, memory)
    return out[0, 0]


def kernel(obs, memory, W, b):
    return _novelty(obs.T, memory, W, b.reshape(-1, 1))


# speedup vs baseline: 2.2764x; 1.5336x over previous
import functools

import jax
import jax.numpy as jnp
from jax.experimental import pallas as pl
from jax.experimental.pallas import tpu as pltpu

TM = 2000
NS = 4
KD = 5


def _body(obsT_ref, W_ref, bT_ref, mem_ref, out_ref, embT_ref, q2_ref, run_ref,
          *, n_tiles, n_q, k_top):
    i = pl.program_id(0)

    @pl.when(i == 0)
    def _init():
        embT = jax.lax.dot_general(
            W_ref[...], obsT_ref[...], (((0,), (0,)), ((), ())),
            preferred_element_type=jnp.float32)
        embT = embT + bT_ref[...]
        q2 = jnp.sum(embT * embT, axis=0, keepdims=True)
        q2_ref[...] = jnp.broadcast_to(q2, q2_ref.shape)
        embT_ref[...] = -2.0 * embT
        run_ref[...] = jnp.full(run_ref.shape, jnp.inf, jnp.float32)

    mem = mem_ref[...]
    qm = jax.lax.dot_general(
        mem, embT_ref[...], (((1,), (0,)), ((), ())),
        preferred_element_type=jnp.float32)
    m2 = jnp.sum(mem * mem, axis=1, keepdims=True)
    s = qm + m2

    stacks = [[run_ref[(st * KD + j) * 8:(st * KD + j) * 8 + 8, :]
               for j in range(KD)] for st in range(NS)]
    n_grp = TM // 8
    for r in range(n_grp):
        t = s[r * 8:r * 8 + 8, :]
        b = stacks[r % NS]
        for j in range(KD):
            lo = jnp.minimum(b[j], t)
            t = jnp.maximum(b[j], t)
            b[j] = lo
    run_ref[...] = jnp.concatenate([stacks[st][j] for st in range(NS)
                                    for j in range(KD)], axis=0)

    @pl.when(i == n_tiles - 1)
    def _fin():
        cand = jnp.concatenate([stacks[st][j] for st in range(NS)
                                for j in range(KD)], axis=0)
        rows = jax.lax.broadcasted_iota(jnp.int32, cand.shape, 0)
        total = jnp.float32(0.0)
        x = cand
        for _ in range(k_top):
            mn = jnp.min(x, axis=0, keepdims=True)
            first = jnp.min(jnp.where(x == mn, rows, x.shape[0]),
                            axis=0, keepdims=True)
            x = jnp.where(rows == first, jnp.inf, x)
            d2 = jnp.maximum(mn + q2_ref[0:1, :], 0.0) + 1e-12
            total = total + jnp.sum(jnp.sqrt(d2))
        out_ref[0, 0] = total / (k_top * n_q)


@jax.jit
def _novelty(obsT, memory, W, bT):
    m_total, d_dim = memory.shape
    n_q = obsT.shape[1]
    k_top = 5
    n_tiles = m_total // TM
    out = pl.pallas_call(
        functools.partial(_body, n_tiles=n_tiles, n_q=n_q, k_top=k_top),
        grid=(n_tiles,),
        in_specs=[
            pl.BlockSpec((obsT.shape[0], n_q), lambda i: (0, 0)),
            pl.BlockSpec(W.shape, lambda i: (0, 0)),
            pl.BlockSpec((d_dim, 1), lambda i: (0, 0)),
            pl.BlockSpec((TM, d_dim), lambda i: (i, 0)),
        ],
        out_specs=pl.BlockSpec(memory_space=pltpu.SMEM),
        out_shape=jax.ShapeDtypeStruct((1, 1), jnp.float32),
        scratch_shapes=[
            pltpu.VMEM((d_dim, n_q), jnp.float32),
            pltpu.VMEM((8, n_q), jnp.float32),
            pltpu.VMEM((NS * KD * 8, n_q), jnp.float32),
        ],
        compiler_params=pltpu.CompilerParams(
            dimension_semantics=("arbitrary",)),
    )(obsT, W, bT, memory)
    return out[0, 0]


def kernel(obs, memory, W, b):
    return _novelty(obs.T, memory, W, b.reshape(-1, 1))


# speedup vs baseline: 2.7250x; 1.1970x over previous
import functools

import jax
import jax.numpy as jnp
from jax.experimental import pallas as pl
from jax.experimental.pallas import tpu as pltpu

TM = 4000
NS = 4
KD = 5


def _body(obsT_ref, W_ref, bT_ref, mem_ref, out_ref, embT_ref, q2_ref, run_ref,
          *, n_tiles, n_q, k_top):
    i = pl.program_id(0)

    @pl.when(i == 0)
    def _init():
        embT = jax.lax.dot_general(
            W_ref[...], obsT_ref[...], (((0,), (0,)), ((), ())),
            preferred_element_type=jnp.float32)
        embT = embT + bT_ref[...]
        q2 = jnp.sum(embT * embT, axis=0, keepdims=True)
        q2_ref[...] = jnp.broadcast_to(q2, q2_ref.shape)
        embT_ref[...] = -2.0 * embT
        run_ref[...] = jnp.full(run_ref.shape, jnp.inf, jnp.float32)

    mem = mem_ref[...]
    qm = jax.lax.dot_general(
        mem, embT_ref[...], (((1,), (0,)), ((), ())),
        preferred_element_type=jnp.float32)
    m2 = jnp.sum(mem * mem, axis=1, keepdims=True)
    s = qm + m2

    stacks = [[run_ref[(st * KD + j) * 8:(st * KD + j) * 8 + 8, :]
               for j in range(KD)] for st in range(NS)]
    n_grp = TM // 8
    for r in range(n_grp):
        t = s[r * 8:r * 8 + 8, :]
        b = stacks[r % NS]
        for j in range(KD):
            lo = jnp.minimum(b[j], t)
            t = jnp.maximum(b[j], t)
            b[j] = lo
    for st in range(NS):
        for j in range(KD):
            base = (st * KD + j) * 8
            run_ref[base:base + 8, :] = stacks[st][j]

    @pl.when(i == n_tiles - 1)
    def _fin():
        cand = jnp.concatenate([stacks[st][j] for st in range(NS)
                                for j in range(KD)], axis=0)
        rows = jax.lax.broadcasted_iota(jnp.int32, cand.shape, 0)
        total = jnp.float32(0.0)
        x = cand
        for _ in range(k_top):
            mn = jnp.min(x, axis=0, keepdims=True)
            first = jnp.min(jnp.where(x == mn, rows, x.shape[0]),
                            axis=0, keepdims=True)
            x = jnp.where(rows == first, jnp.inf, x)
            d2 = jnp.maximum(mn + q2_ref[0:1, :], 0.0) + 1e-12
            total = total + jnp.sum(jnp.sqrt(d2))
        out_ref[0, 0] = total / (k_top * n_q)


@jax.jit
def _novelty(obsT, memory, W, bT):
    m_total, d_dim = memory.shape
    n_q = obsT.shape[1]
    k_top = 5
    n_tiles = m_total // TM
    out = pl.pallas_call(
        functools.partial(_body, n_tiles=n_tiles, n_q=n_q, k_top=k_top),
        grid=(n_tiles,),
        in_specs=[
            pl.BlockSpec((obsT.shape[0], n_q), lambda i: (0, 0)),
            pl.BlockSpec(W.shape, lambda i: (0, 0)),
            pl.BlockSpec((d_dim, 1), lambda i: (0, 0)),
            pl.BlockSpec((TM, d_dim), lambda i: (i, 0)),
        ],
        out_specs=pl.BlockSpec(memory_space=pltpu.SMEM),
        out_shape=jax.ShapeDtypeStruct((1, 1), jnp.float32),
        scratch_shapes=[
            pltpu.VMEM((d_dim, n_q), jnp.float32),
            pltpu.VMEM((8, n_q), jnp.float32),
            pltpu.VMEM((NS * KD * 8, n_q), jnp.float32),
        ],
        compiler_params=pltpu.CompilerParams(
            dimension_semantics=("arbitrary",)),
    )(obsT, W, bT, memory)
    return out[0, 0]


def kernel(obs, memory, W, b):
    return _novelty(obs.T, memory, W, b.reshape(-1, 1))
